# Initial kernel scaffold; baseline (speedup 1.0000x reference)
import jax
import jax.numpy as jnp
from jax.experimental import pallas as pl


def kernel(jd, cv, jd_table, cv_table, W_combine, b_combine, W1, b1, W2, b2):
    raise NotImplementedError("write your pallas kernel here")


# speedup vs baseline: 2.7198x; 2.7198x over previous
import functools

import jax
import jax.numpy as jnp
from jax import lax
from jax.experimental import pallas as pl
from jax.experimental.pallas import tpu as pltpu
from jax.experimental.pallas import tpu_sc as plsc

B = 4096
V = 100000
D = 1536
H = 512

NC = 2
NS = 16
NW = NC * NS
ROWS_PER_W = B // NW
CH = 32
NCHUNK = ROWS_PER_W // CH


def _sc_gather_body(idx_hbm, jd_tab, cv_tab, jd_out, cv_out,
                    idx_v, buf0, buf1, sem0, sem1):
  cid = lax.axis_index("c")
  sid = lax.axis_index("s")
  wid = sid * NC + cid
  pltpu.sync_copy(idx_hbm.at[wid], idx_v)
  base = wid * ROWS_PER_W
  bufs = (buf0, buf1)
  sems = (sem0, sem1)
  work = [(t, ch) for t in range(2) for ch in range(NCHUNK)]
  tabs = (jd_tab, cv_tab)
  outs = (jd_out, cv_out)
  copies = [None, None]
  for k, (t, ch) in enumerate(work):
    p = k % 2
    copies[p] = pltpu.async_copy(tabs[t].at[idx_v.at[t, ch]], bufs[p], sems[p])
    if k >= 1:
      pt, pch = work[k - 1]
      pp = (k - 1) % 2
      copies[pp].wait()
      pltpu.sync_copy(bufs[pp], outs[pt].at[pl.ds(base + pch * CH, CH)])
  lt, lch = work[-1]
  lp = (len(work) - 1) % 2
  copies[lp].wait()
  pltpu.sync_copy(bufs[lp], outs[lt].at[pl.ds(base + lch * CH, CH)])


def _sc_gather(idx, jd_table, cv_table):
  mesh = plsc.VectorSubcoreMesh(core_axis_name="c", subcore_axis_name="s")
  return pl.kernel(
      _sc_gather_body,
      mesh=mesh,
      out_type=[
          jax.ShapeDtypeStruct((B, D), jnp.float32),
          jax.ShapeDtypeStruct((B, D), jnp.float32),
      ],
      scratch_types=[
          pltpu.VMEM((2, NCHUNK, CH), jnp.int32),
          pltpu.VMEM((CH, D), jnp.float32),
          pltpu.VMEM((CH, D), jnp.float32),
          pltpu.SemaphoreType.DMA,
          pltpu.SemaphoreType.DMA,
      ],
  )(idx, jd_table, cv_table)


BM = 512


def _mlp_body(jd_ref, cv_ref, wj_ref, wc_ref, bc_ref, w1_ref, b1_ref,
              w2_ref, b2_ref, out_ref):
  x = (jnp.dot(jd_ref[...], wj_ref[...], preferred_element_type=jnp.float32)
       + jnp.dot(cv_ref[...], wc_ref[...], preferred_element_type=jnp.float32)
       + bc_ref[...])
  x = jnp.where(x >= 0, x, 0.01 * x)
  x = jnp.dot(x, w1_ref[...], preferred_element_type=jnp.float32) + b1_ref[...]
  x = jnp.where(x >= 0, x, 0.01 * x)
  out_ref[...] = (
      jnp.dot(x, w2_ref[...], preferred_element_type=jnp.float32) + b2_ref[...])


def _mlp(jd_e, cv_e, wj, wc, bc, w1, b1, w2, b2):
  grid = (B // BM,)
  return pl.pallas_call(
      _mlp_body,
      grid=grid,
      in_specs=[
          pl.BlockSpec((BM, D), lambda i: (i, 0)),
          pl.BlockSpec((BM, D), lambda i: (i, 0)),
          pl.BlockSpec((D, H), lambda i: (0, 0)),
          pl.BlockSpec((D, H), lambda i: (0, 0)),
          pl.BlockSpec((1, H), lambda i: (0, 0)),
          pl.BlockSpec((H, H), lambda i: (0, 0)),
          pl.BlockSpec((1, H), lambda i: (0, 0)),
          pl.BlockSpec((H, 1), lambda i: (0, 0)),
          pl.BlockSpec((1, 1), lambda i: (0, 0)),
      ],
      out_specs=pl.BlockSpec((BM, 1), lambda i: (i, 0)),
      out_shape=jax.ShapeDtypeStruct((B, 1), jnp.float32),
  )(jd_e, cv_e, wj, wc, bc, w1, b1, w2, b2)


@jax.jit
def kernel(jd, cv, jd_table, cv_table, W_combine, b_combine, W1, b1, W2, b2):
  idx = jnp.stack([jd, cv]).reshape(2, NW, NCHUNK, CH).transpose(1, 0, 2, 3)
  jd_e, cv_e = _sc_gather(idx, jd_table, cv_table)
  wj = W_combine[:D]
  wc = W_combine[D:]
  return _mlp(jd_e, cv_e, wj, wc, b_combine.reshape(1, H), W1,
              b1.reshape(1, H), W2, b2.reshape(1, 1))
